# baseline (speedup 1.0000x reference)
import functools
import jax
import jax.numpy as jnp
from jax import lax
from jax.experimental import pallas as pl
from jax.experimental.pallas import tpu as pltpu
from jax.experimental.pallas import tpu_sc as plsc

N = 10000
NP = 10240
NA = 10112
E = 160000
D = 256
H = 128
NC, NS = 2, 16
CH = 128
EP = 163840
JUNK = 10100
NCHK = EP // NS // CH

RPT = NP // NS
RPA = NA // NS

_MESH = plsc.VectorSubcoreMesh(
    core_axis_name="c", subcore_axis_name="s", num_cores=NC, num_subcores=NS)


def _deg_body(dst_hbm, ones_hbm, zeros_hbm, out_hbm, idx_v, ones_v, acc, sem):
    c = lax.axis_index("c")
    s = lax.axis_index("s")
    w = c * NS + s
    r0 = s * RPT
    pltpu.sync_copy(zeros_hbm, acc.at[pl.ds(r0, RPT)])
    pltpu.sync_copy(dst_hbm.at[w], idx_v)
    pltpu.sync_copy(ones_hbm, ones_v)
    plsc.subcore_barrier()

    def body(j, carry):
        pltpu.sync_copy(ones_v, acc.at[idx_v.at[j]], add=True)
        return carry

    lax.fori_loop(0, dst_hbm.shape[1], body, 0)
    plsc.subcore_barrier()
    pltpu.sync_copy(acc.at[pl.ds(r0, RPT)], out_hbm.at[c].at[pl.ds(r0, RPT)])


_deg_kernel = functools.partial(
    pl.kernel,
    out_type=jax.ShapeDtypeStruct((NC, NP, 16), jnp.float32),
    mesh=_MESH,
    scratch_types=[
        pltpu.VMEM((EP // (NC * NS) // CH, CH), jnp.int32),
        pltpu.VMEM((CH, 16), jnp.float32),
        pltpu.VMEM_SHARED((NP, 16), jnp.float32),
        pltpu.SemaphoreType.DMA,
    ],
)(_deg_body)


def _agg_body(y_hbm, src_hbm, dst_hbm, out_hbm, src_v, dst_v,
              rows_v, acc, sg):
    c = lax.axis_index("c")
    s = lax.axis_index("s")
    r0 = s * RPA
    pltpu.sync_copy(y_hbm.at[c].at[pl.ds(r0, RPA)], acc.at[pl.ds(r0, RPA)])
    pltpu.sync_copy(src_hbm.at[s], src_v)
    pltpu.sync_copy(dst_hbm.at[s], dst_v)
    plsc.subcore_barrier()

    def body(j, carry):
        pltpu.async_copy(y_hbm.at[c].at[src_v.at[j]], rows_v, sg).wait()
        pltpu.sync_copy(rows_v, acc.at[dst_v.at[j]], add=True)
        return carry

    lax.fori_loop(0, NCHK, body, 0)
    plsc.subcore_barrier()
    pltpu.sync_copy(acc.at[pl.ds(r0, RPA)], out_hbm.at[c].at[pl.ds(r0, RPA)])


_agg_kernel = functools.partial(
    pl.kernel,
    out_type=jax.ShapeDtypeStruct((NC, NP, H), jnp.float32),
    mesh=_MESH,
    scratch_types=[
        pltpu.VMEM((NCHK, CH), jnp.int32),
        pltpu.VMEM((NCHK, CH), jnp.int32),
        pltpu.VMEM((CH, H), jnp.float32),
        pltpu.VMEM_SHARED((NA, H), jnp.float32),
        pltpu.SemaphoreType.DMA,
    ],
)(_agg_body)


BLK = 1024


def _dinv_of(deg_ref):
    return lax.rsqrt(deg_ref[0, :, :1] + deg_ref[1, :, :1])


def _mm1_body(x_ref, w_ref, y_ref):
    y_ref[0] = jnp.dot(x_ref[...], w_ref[...],
                       preferred_element_type=jnp.float32)


def _scale_body(z_ref, deg_ref, y_ref):
    y_ref[...] = _dinv_of(deg_ref) * z_ref[...]


def _mm_fused_body(z_ref, w_ref, deg_ref, b_ref, y_ref):
    dinv = _dinv_of(deg_ref)
    z = jnp.concatenate([z_ref[0], z_ref[1]], axis=-1)
    x = jnp.maximum(dinv * z + b_ref[...], 0.0)
    y_ref[0] = dinv * jnp.dot(x, w_ref[...], preferred_element_type=jnp.float32)


def _final_body(z_ref, deg_ref, b_ref, out_ref):
    dinv = _dinv_of(deg_ref)
    z = jnp.concatenate([z_ref[0], z_ref[1]], axis=-1)
    h = dinv * z + b_ref[...]
    m = jnp.max(h, axis=-1, keepdims=True)
    e = jnp.exp(h - m)
    lse = m + jnp.log(jnp.sum(e, axis=-1, keepdims=True))
    out_ref[...] = h - lse


_deg_spec = pl.BlockSpec((2, BLK, 16), lambda i, h: (0, i, 0))
_w_spec = pl.BlockSpec((D, H), lambda i, h: (0, h))
_y_out_spec = pl.BlockSpec((1, BLK, H), lambda i, h: (h, i, 0))
_z_spec = pl.BlockSpec((2, BLK, H), lambda i, h: (0, i, 0))
_b_spec = pl.BlockSpec((1, D), lambda i, h: (0, 0))

_mm1 = pl.pallas_call(
    _mm1_body,
    grid=(NP // BLK, 2),
    in_specs=[pl.BlockSpec((BLK, D), lambda i, h: (i, 0)), _w_spec],
    out_specs=_y_out_spec,
    out_shape=jax.ShapeDtypeStruct((NC, NP, H), jnp.float32),
)

_scale = pl.pallas_call(
    _scale_body,
    grid=(NP // BLK, 2),
    in_specs=[pl.BlockSpec((1, BLK, H), lambda i, h: (h, i, 0)), _deg_spec],
    out_specs=_y_out_spec,
    out_shape=jax.ShapeDtypeStruct((NC, NP, H), jnp.float32),
)

_mm_fused = pl.pallas_call(
    _mm_fused_body,
    grid=(NP // BLK, 2),
    in_specs=[_z_spec, _w_spec, _deg_spec, _b_spec],
    out_specs=_y_out_spec,
    out_shape=jax.ShapeDtypeStruct((NC, NP, H), jnp.float32),
)

_final = pl.pallas_call(
    _final_body,
    grid=(NP // BLK,),
    in_specs=[
        pl.BlockSpec((2, BLK, H), lambda i: (0, i, 0)),
        pl.BlockSpec((2, BLK, 16), lambda i: (0, i, 0)),
        pl.BlockSpec((1, D), lambda i: (0, 0)),
    ],
    out_specs=pl.BlockSpec((BLK, D), lambda i: (i, 0)),
    out_shape=jax.ShapeDtypeStruct((N, D), jnp.float32),
)


@jax.jit
def kernel(graph, nfeat, W1, b1, W2, b2, W3, b3):
    src = graph[0].astype(jnp.int32)
    dst = graph[1].astype(jnp.int32)
    pad = EP - E
    src_p = jnp.concatenate([src, jnp.zeros((pad,), jnp.int32)])
    dst_p = jnp.concatenate([dst, jnp.full((pad,), JUNK, jnp.int32)])
    dst_deg = dst_p.reshape(NC * NS, EP // (NC * NS) // CH, CH)
    src_agg = src_p.reshape(NS, NCHK, CH)
    dst_agg = dst_p.reshape(NS, NCHK, CH)
    ones_c = jnp.ones((CH, 16), jnp.float32)
    zeros_c = jnp.zeros((RPT, 16), jnp.float32)
    x0 = jnp.pad(nfeat, ((0, NP - N), (0, 0)))

    deg = _deg_kernel(dst_deg, ones_c, zeros_c)

    xw = _mm1(x0, W1)
    y = _scale(xw, deg)
    z = _agg_kernel(y, src_agg, dst_agg)
    y = _mm_fused(z, W2, deg, b1.reshape(1, D))
    z = _agg_kernel(y, src_agg, dst_agg)
    y = _mm_fused(z, W3, deg, b2.reshape(1, D))
    z = _agg_kernel(y, src_agg, dst_agg)
    return _final(z, deg, b3.reshape(1, D))


# speedup vs baseline: 1.6733x; 1.6733x over previous
import functools
import jax
import jax.numpy as jnp
from jax import lax
from jax.experimental import pallas as pl
from jax.experimental.pallas import tpu as pltpu
from jax.experimental.pallas import tpu_sc as plsc

N = 10000
NP = 10240
E = 160000
D = 256
H = 128
NC, NS = 2, 16
CH = 125
RPT = NP // NS

_MESH = plsc.VectorSubcoreMesh(
    core_axis_name="c", subcore_axis_name="s", num_cores=NC, num_subcores=NS)


def _deg_body(dst_hbm, ones_hbm, zeros_hbm, out_hbm, idx_v, ones_v, acc, sem):
    c = lax.axis_index("c")
    s = lax.axis_index("s")
    w = c * NS + s
    r0 = s * RPT
    pltpu.sync_copy(zeros_hbm, acc.at[pl.ds(r0, RPT)])
    pltpu.sync_copy(dst_hbm.at[w], idx_v)
    pltpu.sync_copy(ones_hbm, ones_v)
    plsc.subcore_barrier()

    def body(j, carry):
        pltpu.sync_copy(ones_v, acc.at[idx_v.at[j]], add=True)
        return carry

    lax.fori_loop(0, dst_hbm.shape[1], body, 0)
    plsc.subcore_barrier()
    pltpu.sync_copy(acc.at[pl.ds(r0, RPT)], out_hbm.at[c].at[pl.ds(r0, RPT)])


_deg_kernel = functools.partial(
    pl.kernel,
    out_type=jax.ShapeDtypeStruct((NC, NP, 16), jnp.float32),
    mesh=_MESH,
    scratch_types=[
        pltpu.VMEM((E // (NC * NS) // CH, CH), jnp.int32),
        pltpu.VMEM((CH, 16), jnp.float32),
        pltpu.VMEM_SHARED((NP, 16), jnp.float32),
        pltpu.SemaphoreType.DMA,
    ],
)(_deg_body)


def _agg_body(y_hbm, src_hbm, dst_hbm, out_hbm, src_v, dst_v, rows_v, acc, sem):
    c = lax.axis_index("c")
    s = lax.axis_index("s")
    r0 = s * RPT
    pltpu.sync_copy(y_hbm.at[c].at[pl.ds(r0, RPT)], acc.at[pl.ds(r0, RPT)])
    pltpu.sync_copy(src_hbm.at[s], src_v)
    pltpu.sync_copy(dst_hbm.at[s], dst_v)
    plsc.subcore_barrier()

    def body(j, carry):
        pltpu.async_copy(y_hbm.at[c].at[src_v.at[j]], rows_v, sem).wait()
        pltpu.sync_copy(rows_v, acc.at[dst_v.at[j]], add=True)
        return carry

    lax.fori_loop(0, src_hbm.shape[1], body, 0)
    plsc.subcore_barrier()
    pltpu.sync_copy(acc.at[pl.ds(r0, RPT)], out_hbm.at[c].at[pl.ds(r0, RPT)])


_agg_kernel = functools.partial(
    pl.kernel,
    out_type=jax.ShapeDtypeStruct((NC, NP, H), jnp.float32),
    mesh=_MESH,
    scratch_types=[
        pltpu.VMEM((E // NS // CH, CH), jnp.int32),
        pltpu.VMEM((E // NS // CH, CH), jnp.int32),
        pltpu.VMEM((CH, H), jnp.float32),
        pltpu.VMEM_SHARED((NP, H), jnp.float32),
        pltpu.SemaphoreType.DMA,
    ],
)(_agg_body)


BLK = 1024


def _dinv_of(deg_ref):
    return lax.rsqrt(deg_ref[0, :, :1] + deg_ref[1, :, :1])


def _mm1_body(x_ref, w_ref, y_ref):
    y_ref[0] = jnp.dot(x_ref[...], w_ref[...],
                       preferred_element_type=jnp.float32)


def _scale_body(z_ref, deg_ref, y_ref):
    y_ref[...] = _dinv_of(deg_ref) * z_ref[...]


def _mm_fused_body(z_ref, w_ref, deg_ref, b_ref, y_ref):
    dinv = _dinv_of(deg_ref)
    z = jnp.concatenate([z_ref[0], z_ref[1]], axis=-1)
    x = jnp.maximum(dinv * z + b_ref[...], 0.0)
    y_ref[0] = dinv * jnp.dot(x, w_ref[...],
                              preferred_element_type=jnp.float32)


def _final_body(z_ref, deg_ref, b_ref, out_ref):
    dinv = _dinv_of(deg_ref)
    z = jnp.concatenate([z_ref[0], z_ref[1]], axis=-1)
    h = dinv * z + b_ref[...]
    m = jnp.max(h, axis=-1, keepdims=True)
    e = jnp.exp(h - m)
    lse = m + jnp.log(jnp.sum(e, axis=-1, keepdims=True))
    out_ref[...] = h - lse


_deg_spec = pl.BlockSpec((2, BLK, 16), lambda i, h: (0, i, 0))
_w_spec = pl.BlockSpec((D, H), lambda i, h: (0, h))
_y_out_spec = pl.BlockSpec((1, BLK, H), lambda i, h: (h, i, 0))
_z_spec = pl.BlockSpec((2, BLK, H), lambda i, h: (0, i, 0))
_b_spec = pl.BlockSpec((1, D), lambda i, h: (0, 0))

_mm1 = pl.pallas_call(
    _mm1_body,
    grid=(NP // BLK, 2),
    in_specs=[pl.BlockSpec((BLK, D), lambda i, h: (i, 0)), _w_spec],
    out_specs=_y_out_spec,
    out_shape=jax.ShapeDtypeStruct((NC, NP, H), jnp.float32),
)

_scale = pl.pallas_call(
    _scale_body,
    grid=(NP // BLK, 2),
    in_specs=[pl.BlockSpec((1, BLK, H), lambda i, h: (h, i, 0)), _deg_spec],
    out_specs=_y_out_spec,
    out_shape=jax.ShapeDtypeStruct((NC, NP, H), jnp.float32),
)

_mm_fused = pl.pallas_call(
    _mm_fused_body,
    grid=(NP // BLK, 2),
    in_specs=[_z_spec, _w_spec, _deg_spec, _b_spec],
    out_specs=_y_out_spec,
    out_shape=jax.ShapeDtypeStruct((NC, NP, H), jnp.float32),
)

_final = pl.pallas_call(
    _final_body,
    grid=(NP // BLK,),
    in_specs=[
        pl.BlockSpec((2, BLK, H), lambda i: (0, i, 0)),
        pl.BlockSpec((2, BLK, 16), lambda i: (0, i, 0)),
        pl.BlockSpec((1, D), lambda i: (0, 0)),
    ],
    out_specs=pl.BlockSpec((BLK, D), lambda i: (i, 0)),
    out_shape=jax.ShapeDtypeStruct((N, D), jnp.float32),
)


@jax.jit
def kernel(graph, nfeat, W1, b1, W2, b2, W3, b3):
    src = graph[0].astype(jnp.int32)
    dst = graph[1].astype(jnp.int32)
    dst_deg = dst.reshape(NC * NS, E // (NC * NS) // CH, CH)
    src_agg = src.reshape(NS, E // NS // CH, CH)
    dst_agg = dst.reshape(NS, E // NS // CH, CH)
    ones_c = jnp.ones((CH, 16), jnp.float32)
    zeros_c = jnp.zeros((RPT, 16), jnp.float32)
    x0 = jnp.pad(nfeat, ((0, NP - N), (0, 0)))

    deg = _deg_kernel(dst_deg, ones_c, zeros_c)

    xw = _mm1(x0, W1)
    y = _scale(xw, deg)
    z = _agg_kernel(y, src_agg, dst_agg)
    y = _mm_fused(z, W2, deg, b1.reshape(1, D))
    z = _agg_kernel(y, src_agg, dst_agg)
    y = _mm_fused(z, W3, deg, b2.reshape(1, D))
    z = _agg_kernel(y, src_agg, dst_agg)
    return _final(z, deg, b3.reshape(1, D))


# speedup vs baseline: 2.0454x; 1.2224x over previous
import functools
import jax
import jax.numpy as jnp
from jax import lax
from jax.experimental import pallas as pl
from jax.experimental.pallas import tpu as pltpu
from jax.experimental.pallas import tpu_sc as plsc

N = 10000
NP = 10240
E = 160000
D = 256
H = 128
NC, NS = 2, 16
CH = 125
RPT = NP // NS

_MESH = plsc.VectorSubcoreMesh(
    core_axis_name="c", subcore_axis_name="s", num_cores=NC, num_subcores=NS)


def _deg_body(dst_hbm, ones_hbm, zeros_hbm, out_hbm, idx_v, ones_v, acc, sem):
    c = lax.axis_index("c")
    s = lax.axis_index("s")
    w = c * NS + s
    r0 = s * RPT
    pltpu.sync_copy(zeros_hbm, acc.at[pl.ds(r0, RPT)])
    pltpu.sync_copy(dst_hbm.at[w], idx_v)
    pltpu.sync_copy(ones_hbm, ones_v)
    plsc.subcore_barrier()

    def body(j, carry):
        pltpu.sync_copy(ones_v, acc.at[idx_v.at[j]], add=True)
        return carry

    lax.fori_loop(0, dst_hbm.shape[1], body, 0)
    plsc.subcore_barrier()
    pltpu.sync_copy(acc.at[pl.ds(r0, RPT)], out_hbm.at[c].at[pl.ds(r0, RPT)])


_deg_kernel = functools.partial(
    pl.kernel,
    out_type=jax.ShapeDtypeStruct((NC, NP, 16), jnp.float32),
    mesh=_MESH,
    scratch_types=[
        pltpu.VMEM((E // (NC * NS) // CH, CH), jnp.int32),
        pltpu.VMEM((CH, 16), jnp.float32),
        pltpu.VMEM_SHARED((NP, 16), jnp.float32),
        pltpu.SemaphoreType.DMA,
    ],
)(_deg_body)


UNROLL = 8
NB = (E // NS // CH) // UNROLL


def _agg_body(y_hbm, src_hbm, dst_hbm, out_hbm, src_v, dst_v,
              rows0, rows1, acc, sg0, sg1, sd):
    c = lax.axis_index("c")
    s = lax.axis_index("s")
    r0 = s * RPT
    pltpu.sync_copy(y_hbm.at[c].at[pl.ds(r0, RPT)], acc.at[pl.ds(r0, RPT)])
    pltpu.sync_copy(src_hbm.at[s], src_v)
    plsc.subcore_barrier()

    rbufs = (rows0, rows1)
    gsems = (sg0, sg1)

    def gather(j, k):
        pltpu.async_copy(y_hbm.at[c].at[src_v.at[j]], rbufs[k % 2],
                         gsems[k % 2])

    def wait_gather(j, k):
        pltpu.make_async_copy(y_hbm.at[c].at[src_v.at[j]], rbufs[k % 2],
                              gsems[k % 2]).wait()

    def dst_half(g):
        return dst_v.at[pl.ds(lax.rem(g, 2) * UNROLL, UNROLL)]

    def dst_src(g):
        return dst_hbm.at[s].at[pl.ds(g * UNROLL, UNROLL)]

    pltpu.async_copy(dst_src(0), dst_half(0), sd)

    def body(g, carry):
        base = g * UNROLL
        pltpu.make_async_copy(dst_src(g), dst_half(g), sd).wait()

        @pl.when(g + 1 < NB)
        def _():
            pltpu.async_copy(dst_src(g + 1), dst_half(g + 1), sd)

        gather(base, 0)
        for k in range(UNROLL):
            j = base + k
            wait_gather(j, k)
            if k + 1 < UNROLL:
                gather(j + 1, k + 1)
            pltpu.sync_copy(rbufs[k % 2],
                            acc.at[dst_v.at[lax.rem(g, 2) * UNROLL + k]],
                            add=True)
        return carry

    lax.fori_loop(0, NB, body, 0)
    plsc.subcore_barrier()
    pltpu.sync_copy(acc.at[pl.ds(r0, RPT)], out_hbm.at[c].at[pl.ds(r0, RPT)])


_agg_kernel = functools.partial(
    pl.kernel,
    out_type=jax.ShapeDtypeStruct((NC, NP, H), jnp.float32),
    mesh=_MESH,
    scratch_types=[
        pltpu.VMEM((E // NS // CH, CH), jnp.int32),
        pltpu.VMEM((2 * UNROLL, CH), jnp.int32),
        pltpu.VMEM((CH, H), jnp.float32),
        pltpu.VMEM((CH, H), jnp.float32),
        pltpu.VMEM_SHARED((NP, H), jnp.float32),
        pltpu.SemaphoreType.DMA,
        pltpu.SemaphoreType.DMA,
        pltpu.SemaphoreType.DMA,
    ],
)(_agg_body)


BLK = 1024


def _dinv_of(deg_ref):
    return lax.rsqrt(deg_ref[0, :, :1] + deg_ref[1, :, :1])


def _mm1_body(x_ref, w_ref, y_ref):
    y_ref[0] = jnp.dot(x_ref[...], w_ref[...],
                       preferred_element_type=jnp.float32)


def _scale_body(z_ref, deg_ref, y_ref):
    y_ref[...] = _dinv_of(deg_ref) * z_ref[...]


def _mm_fused_body(z_ref, w_ref, deg_ref, b_ref, y_ref):
    dinv = _dinv_of(deg_ref)
    z = jnp.concatenate([z_ref[0], z_ref[1]], axis=-1)
    x = jnp.maximum(dinv * z + b_ref[...], 0.0)
    y_ref[0] = dinv * jnp.dot(x, w_ref[...],
                              preferred_element_type=jnp.float32)


def _final_body(z_ref, deg_ref, b_ref, out_ref):
    dinv = _dinv_of(deg_ref)
    z = jnp.concatenate([z_ref[0], z_ref[1]], axis=-1)
    h = dinv * z + b_ref[...]
    m = jnp.max(h, axis=-1, keepdims=True)
    e = jnp.exp(h - m)
    lse = m + jnp.log(jnp.sum(e, axis=-1, keepdims=True))
    out_ref[...] = h - lse


_deg_spec = pl.BlockSpec((2, BLK, 16), lambda i, h: (0, i, 0))
_w_spec = pl.BlockSpec((D, H), lambda i, h: (0, h))
_y_out_spec = pl.BlockSpec((1, BLK, H), lambda i, h: (h, i, 0))
_z_spec = pl.BlockSpec((2, BLK, H), lambda i, h: (0, i, 0))
_b_spec = pl.BlockSpec((1, D), lambda i, h: (0, 0))

_mm1 = pl.pallas_call(
    _mm1_body,
    grid=(NP // BLK, 2),
    in_specs=[pl.BlockSpec((BLK, D), lambda i, h: (i, 0)), _w_spec],
    out_specs=_y_out_spec,
    out_shape=jax.ShapeDtypeStruct((NC, NP, H), jnp.float32),
)

_scale = pl.pallas_call(
    _scale_body,
    grid=(NP // BLK, 2),
    in_specs=[pl.BlockSpec((1, BLK, H), lambda i, h: (h, i, 0)), _deg_spec],
    out_specs=_y_out_spec,
    out_shape=jax.ShapeDtypeStruct((NC, NP, H), jnp.float32),
)

_mm_fused = pl.pallas_call(
    _mm_fused_body,
    grid=(NP // BLK, 2),
    in_specs=[_z_spec, _w_spec, _deg_spec, _b_spec],
    out_specs=_y_out_spec,
    out_shape=jax.ShapeDtypeStruct((NC, NP, H), jnp.float32),
)

_final = pl.pallas_call(
    _final_body,
    grid=(NP // BLK,),
    in_specs=[
        pl.BlockSpec((2, BLK, H), lambda i: (0, i, 0)),
        pl.BlockSpec((2, BLK, 16), lambda i: (0, i, 0)),
        pl.BlockSpec((1, D), lambda i: (0, 0)),
    ],
    out_specs=pl.BlockSpec((BLK, D), lambda i: (i, 0)),
    out_shape=jax.ShapeDtypeStruct((N, D), jnp.float32),
)


@jax.jit
def kernel(graph, nfeat, W1, b1, W2, b2, W3, b3):
    src = graph[0].astype(jnp.int32)
    dst = graph[1].astype(jnp.int32)
    dst_deg = dst.reshape(NC * NS, E // (NC * NS) // CH, CH)
    src_agg = src.reshape(NS, E // NS // CH, CH)
    dst_agg = dst.reshape(NS, E // NS // CH, CH)
    ones_c = jnp.ones((CH, 16), jnp.float32)
    zeros_c = jnp.zeros((RPT, 16), jnp.float32)
    x0 = jnp.pad(nfeat, ((0, NP - N), (0, 0)))

    deg = _deg_kernel(dst_deg, ones_c, zeros_c)

    xw = _mm1(x0, W1)
    y = _scale(xw, deg)
    z = _agg_kernel(y, src_agg, dst_agg)
    y = _mm_fused(z, W2, deg, b1.reshape(1, D))
    z = _agg_kernel(y, src_agg, dst_agg)
    y = _mm_fused(z, W3, deg, b2.reshape(1, D))
    z = _agg_kernel(y, src_agg, dst_agg)
    return _final(z, deg, b3.reshape(1, D))


# speedup vs baseline: 2.0786x; 1.0162x over previous
import functools
import jax
import jax.numpy as jnp
from jax import lax
from jax.experimental import pallas as pl
from jax.experimental.pallas import tpu as pltpu
from jax.experimental.pallas import tpu_sc as plsc

N = 10000
NP = 10240
E = 160000
D = 256
H = 128
NC, NS = 2, 16
CH = 125
RPT = NP // NS

_MESH = plsc.VectorSubcoreMesh(
    core_axis_name="c", subcore_axis_name="s", num_cores=NC, num_subcores=NS)


def _deg_body(dst_hbm, ones_hbm, zeros_hbm, out_hbm, idx_v, ones_v, acc, sem):
    c = lax.axis_index("c")
    s = lax.axis_index("s")
    w = c * NS + s
    r0 = s * RPT
    pltpu.sync_copy(zeros_hbm, acc.at[pl.ds(r0, RPT)])
    pltpu.sync_copy(dst_hbm.at[w], idx_v)
    pltpu.sync_copy(ones_hbm, ones_v)
    plsc.subcore_barrier()

    def body(j, carry):
        pltpu.sync_copy(ones_v, acc.at[idx_v.at[j]], add=True)
        return carry

    lax.fori_loop(0, dst_hbm.shape[1], body, 0)
    plsc.subcore_barrier()
    pltpu.sync_copy(acc.at[pl.ds(r0, RPT)], out_hbm.at[c].at[pl.ds(r0, RPT)])


_deg_kernel = functools.partial(
    pl.kernel,
    out_type=jax.ShapeDtypeStruct((NC, NP, 16), jnp.float32),
    mesh=_MESH,
    scratch_types=[
        pltpu.VMEM((E // (NC * NS) // CH, CH), jnp.int32),
        pltpu.VMEM((CH, 16), jnp.float32),
        pltpu.VMEM_SHARED((NP, 16), jnp.float32),
        pltpu.SemaphoreType.DMA,
    ],
)(_deg_body)


UNROLL = 16
NB = (E // NS // CH) // UNROLL


def _agg_body(y_hbm, src_hbm, dst_hbm, out_hbm, src_v, dst_v,
              rows0, rows1, acc, sg0, sg1, sd):
    c = lax.axis_index("c")
    s = lax.axis_index("s")
    r0 = s * RPT
    pltpu.sync_copy(y_hbm.at[c].at[pl.ds(r0, RPT)], acc.at[pl.ds(r0, RPT)])
    pltpu.sync_copy(src_hbm.at[s], src_v)
    plsc.subcore_barrier()

    rbufs = (rows0, rows1)
    gsems = (sg0, sg1)

    def gather(j, k):
        pltpu.async_copy(y_hbm.at[c].at[src_v.at[j]], rbufs[k % 2],
                         gsems[k % 2])

    def wait_gather(j, k):
        pltpu.make_async_copy(y_hbm.at[c].at[src_v.at[j]], rbufs[k % 2],
                              gsems[k % 2]).wait()

    def dst_half(g):
        return dst_v.at[pl.ds(lax.rem(g, 2) * UNROLL, UNROLL)]

    def dst_src(g):
        return dst_hbm.at[s].at[pl.ds(g * UNROLL, UNROLL)]

    pltpu.async_copy(dst_src(0), dst_half(0), sd)

    def body(g, carry):
        base = g * UNROLL
        gather(base, 0)
        pltpu.make_async_copy(dst_src(g), dst_half(g), sd).wait()

        @pl.when(g + 1 < NB)
        def _():
            pltpu.async_copy(dst_src(g + 1), dst_half(g + 1), sd)

        for k in range(UNROLL):
            j = base + k
            wait_gather(j, k)
            if k + 1 < UNROLL:
                gather(j + 1, k + 1)
            pltpu.sync_copy(rbufs[k % 2],
                            acc.at[dst_v.at[lax.rem(g, 2) * UNROLL + k]],
                            add=True)
        return carry

    lax.fori_loop(0, NB, body, 0)
    plsc.subcore_barrier()
    pltpu.sync_copy(acc.at[pl.ds(r0, RPT)], out_hbm.at[c].at[pl.ds(r0, RPT)])


_agg_kernel = functools.partial(
    pl.kernel,
    out_type=jax.ShapeDtypeStruct((NC, NP, H), jnp.float32),
    mesh=_MESH,
    scratch_types=[
        pltpu.VMEM((E // NS // CH, CH), jnp.int32),
        pltpu.VMEM((2 * UNROLL, CH), jnp.int32),
        pltpu.VMEM((CH, H), jnp.float32),
        pltpu.VMEM((CH, H), jnp.float32),
        pltpu.VMEM_SHARED((NP, H), jnp.float32),
        pltpu.SemaphoreType.DMA,
        pltpu.SemaphoreType.DMA,
        pltpu.SemaphoreType.DMA,
    ],
)(_agg_body)


BLK = 1024


def _dinv_of(deg_ref):
    return lax.rsqrt(deg_ref[0, :, :1] + deg_ref[1, :, :1])


def _mm1_body(x_ref, w_ref, y_ref):
    y_ref[0] = jnp.dot(x_ref[...], w_ref[...],
                       preferred_element_type=jnp.float32)


def _scale_body(z_ref, deg_ref, y_ref):
    y_ref[...] = _dinv_of(deg_ref) * z_ref[...]


def _mm_fused_body(z_ref, w_ref, deg_ref, b_ref, y_ref):
    dinv = _dinv_of(deg_ref)
    z = jnp.concatenate([z_ref[0], z_ref[1]], axis=-1)
    x = jnp.maximum(dinv * z + b_ref[...], 0.0)
    y_ref[0] = dinv * jnp.dot(x, w_ref[...],
                              preferred_element_type=jnp.float32)


def _final_body(z_ref, deg_ref, b_ref, out_ref):
    dinv = _dinv_of(deg_ref)
    z = jnp.concatenate([z_ref[0], z_ref[1]], axis=-1)
    h = dinv * z + b_ref[...]
    m = jnp.max(h, axis=-1, keepdims=True)
    e = jnp.exp(h - m)
    lse = m + jnp.log(jnp.sum(e, axis=-1, keepdims=True))
    out_ref[...] = h - lse


_deg_spec = pl.BlockSpec((2, BLK, 16), lambda i, h: (0, i, 0))
_w_spec = pl.BlockSpec((D, H), lambda i, h: (0, h))
_y_out_spec = pl.BlockSpec((1, BLK, H), lambda i, h: (h, i, 0))
_z_spec = pl.BlockSpec((2, BLK, H), lambda i, h: (0, i, 0))
_b_spec = pl.BlockSpec((1, D), lambda i, h: (0, 0))

_mm1 = pl.pallas_call(
    _mm1_body,
    grid=(NP // BLK, 2),
    in_specs=[pl.BlockSpec((BLK, D), lambda i, h: (i, 0)), _w_spec],
    out_specs=_y_out_spec,
    out_shape=jax.ShapeDtypeStruct((NC, NP, H), jnp.float32),
)

_scale = pl.pallas_call(
    _scale_body,
    grid=(NP // BLK, 2),
    in_specs=[pl.BlockSpec((1, BLK, H), lambda i, h: (h, i, 0)), _deg_spec],
    out_specs=_y_out_spec,
    out_shape=jax.ShapeDtypeStruct((NC, NP, H), jnp.float32),
)

_mm_fused = pl.pallas_call(
    _mm_fused_body,
    grid=(NP // BLK, 2),
    in_specs=[_z_spec, _w_spec, _deg_spec, _b_spec],
    out_specs=_y_out_spec,
    out_shape=jax.ShapeDtypeStruct((NC, NP, H), jnp.float32),
)

_final = pl.pallas_call(
    _final_body,
    grid=(NP // BLK,),
    in_specs=[
        pl.BlockSpec((2, BLK, H), lambda i: (0, i, 0)),
        pl.BlockSpec((2, BLK, 16), lambda i: (0, i, 0)),
        pl.BlockSpec((1, D), lambda i: (0, 0)),
    ],
    out_specs=pl.BlockSpec((BLK, D), lambda i: (i, 0)),
    out_shape=jax.ShapeDtypeStruct((N, D), jnp.float32),
)


@jax.jit
def kernel(graph, nfeat, W1, b1, W2, b2, W3, b3):
    src = graph[0].astype(jnp.int32)
    dst = graph[1].astype(jnp.int32)
    dst_deg = dst.reshape(NC * NS, E // (NC * NS) // CH, CH)
    src_agg = src.reshape(NS, E // NS // CH, CH)
    dst_agg = dst.reshape(NS, E // NS // CH, CH)
    ones_c = jnp.ones((CH, 16), jnp.float32)
    zeros_c = jnp.zeros((RPT, 16), jnp.float32)
    x0 = jnp.pad(nfeat, ((0, NP - N), (0, 0)))

    deg = _deg_kernel(dst_deg, ones_c, zeros_c)

    xw = _mm1(x0, W1)
    y = _scale(xw, deg)
    z = _agg_kernel(y, src_agg, dst_agg)
    y = _mm_fused(z, W2, deg, b1.reshape(1, D))
    z = _agg_kernel(y, src_agg, dst_agg)
    y = _mm_fused(z, W3, deg, b2.reshape(1, D))
    z = _agg_kernel(y, src_agg, dst_agg)
    return _final(z, deg, b3.reshape(1, D))


# speedup vs baseline: 2.3323x; 1.1221x over previous
import functools
import jax
import jax.numpy as jnp
from jax import lax
from jax.experimental import pallas as pl
from jax.experimental.pallas import tpu as pltpu
from jax.experimental.pallas import tpu_sc as plsc

N = 10000
NP = 10240
E = 160000
D = 256
H = 128
NC, NS = 2, 16
CH = 125
RPT = NP // NS

_MESH = plsc.VectorSubcoreMesh(
    core_axis_name="c", subcore_axis_name="s", num_cores=NC, num_subcores=NS)


def _deg_body(dst_hbm, ones_hbm, zeros_hbm, out_hbm, idx_v, ones_v, acc, sem):
    c = lax.axis_index("c")
    s = lax.axis_index("s")
    w = c * NS + s
    r0 = s * RPT
    pltpu.sync_copy(zeros_hbm, acc.at[pl.ds(r0, RPT)])
    pltpu.sync_copy(dst_hbm.at[w], idx_v)
    pltpu.sync_copy(ones_hbm, ones_v)
    plsc.subcore_barrier()

    def body(j, carry):
        pltpu.sync_copy(ones_v, acc.at[idx_v.at[j]], add=True)
        return carry

    lax.fori_loop(0, dst_hbm.shape[1], body, 0)
    plsc.subcore_barrier()
    pltpu.sync_copy(acc.at[pl.ds(r0, RPT)], out_hbm.at[c].at[pl.ds(r0, RPT)])


_deg_kernel = functools.partial(
    pl.kernel,
    out_type=jax.ShapeDtypeStruct((NC, NP, 16), jnp.float32),
    mesh=_MESH,
    scratch_types=[
        pltpu.VMEM((E // (NC * NS) // CH, CH), jnp.int32),
        pltpu.VMEM((CH, 16), jnp.float32),
        pltpu.VMEM_SHARED((NP, 16), jnp.float32),
        pltpu.SemaphoreType.DMA,
    ],
)(_deg_body)


UNROLL = 16
NB = (E // NS // CH) // UNROLL


def _agg_body(y_hbm, src_hbm, dst_hbm, out_hbm, src_v, dst_v,
              rows0, rows1, acc, sg0, sg1, sd):
    c = lax.axis_index("c")
    s = lax.axis_index("s")
    r0 = s * RPT
    pltpu.sync_copy(y_hbm.at[c].at[pl.ds(r0, RPT)], acc.at[pl.ds(r0, RPT)])
    pltpu.sync_copy(src_hbm.at[s], src_v)
    plsc.subcore_barrier()

    rbufs = (rows0, rows1)
    gsems = (sg0, sg1)

    def gather(j, k):
        pltpu.async_copy(y_hbm.at[c].at[src_v.at[j]], rbufs[k % 2],
                         gsems[k % 2])

    def wait_gather(j, k):
        pltpu.make_async_copy(y_hbm.at[c].at[src_v.at[j]], rbufs[k % 2],
                              gsems[k % 2]).wait()

    def dst_half(g):
        return dst_v.at[pl.ds(lax.rem(g, 2) * UNROLL, UNROLL)]

    def dst_src(g):
        return dst_hbm.at[s].at[pl.ds(g * UNROLL, UNROLL)]

    pltpu.async_copy(dst_src(0), dst_half(0), sd)

    def body(g, carry):
        base = g * UNROLL
        gather(base, 0)
        pltpu.make_async_copy(dst_src(g), dst_half(g), sd).wait()

        @pl.when(g + 1 < NB)
        def _():
            pltpu.async_copy(dst_src(g + 1), dst_half(g + 1), sd)

        for k in range(UNROLL):
            j = base + k
            if k + 1 < UNROLL:
                gather(j + 1, k + 1)
            wait_gather(j, k)
            pltpu.sync_copy(rbufs[k % 2],
                            acc.at[dst_v.at[lax.rem(g, 2) * UNROLL + k]],
                            add=True)
        return carry

    lax.fori_loop(0, NB, body, 0)
    plsc.subcore_barrier()
    pltpu.sync_copy(acc.at[pl.ds(r0, RPT)], out_hbm.at[c].at[pl.ds(r0, RPT)])


_agg_kernel = functools.partial(
    pl.kernel,
    out_type=jax.ShapeDtypeStruct((NC, NP, H), jnp.float32),
    mesh=_MESH,
    scratch_types=[
        pltpu.VMEM((E // NS // CH, CH), jnp.int32),
        pltpu.VMEM((2 * UNROLL, CH), jnp.int32),
        pltpu.VMEM((CH, H), jnp.float32),
        pltpu.VMEM((CH, H), jnp.float32),
        pltpu.VMEM_SHARED((NP, H), jnp.float32),
        pltpu.SemaphoreType.DMA,
        pltpu.SemaphoreType.DMA,
        pltpu.SemaphoreType.DMA,
    ],
)(_agg_body)


BLK = 1024


def _dinv_of(deg_ref):
    return lax.rsqrt(deg_ref[0, :, :1] + deg_ref[1, :, :1])


def _mm1_body(x_ref, w_ref, y_ref):
    y_ref[0] = jnp.dot(x_ref[...], w_ref[...],
                       preferred_element_type=jnp.float32)


def _scale_body(z_ref, deg_ref, y_ref):
    y_ref[...] = _dinv_of(deg_ref) * z_ref[...]


def _mm_fused_body(z_ref, w_ref, deg_ref, b_ref, y_ref):
    dinv = _dinv_of(deg_ref)
    z = jnp.concatenate([z_ref[0], z_ref[1]], axis=-1)
    x = jnp.maximum(dinv * z + b_ref[...], 0.0)
    y_ref[0] = dinv * jnp.dot(x, w_ref[...],
                              preferred_element_type=jnp.float32)


def _final_body(z_ref, deg_ref, b_ref, out_ref):
    dinv = _dinv_of(deg_ref)
    z = jnp.concatenate([z_ref[0], z_ref[1]], axis=-1)
    h = dinv * z + b_ref[...]
    m = jnp.max(h, axis=-1, keepdims=True)
    e = jnp.exp(h - m)
    lse = m + jnp.log(jnp.sum(e, axis=-1, keepdims=True))
    out_ref[...] = h - lse


_deg_spec = pl.BlockSpec((2, BLK, 16), lambda i, h: (0, i, 0))
_w_spec = pl.BlockSpec((D, H), lambda i, h: (0, h))
_y_out_spec = pl.BlockSpec((1, BLK, H), lambda i, h: (h, i, 0))
_z_spec = pl.BlockSpec((2, BLK, H), lambda i, h: (0, i, 0))
_b_spec = pl.BlockSpec((1, D), lambda i, h: (0, 0))

_mm1 = pl.pallas_call(
    _mm1_body,
    grid=(NP // BLK, 2),
    in_specs=[pl.BlockSpec((BLK, D), lambda i, h: (i, 0)), _w_spec],
    out_specs=_y_out_spec,
    out_shape=jax.ShapeDtypeStruct((NC, NP, H), jnp.float32),
)

_scale = pl.pallas_call(
    _scale_body,
    grid=(NP // BLK, 2),
    in_specs=[pl.BlockSpec((1, BLK, H), lambda i, h: (h, i, 0)), _deg_spec],
    out_specs=_y_out_spec,
    out_shape=jax.ShapeDtypeStruct((NC, NP, H), jnp.float32),
)

_mm_fused = pl.pallas_call(
    _mm_fused_body,
    grid=(NP // BLK, 2),
    in_specs=[_z_spec, _w_spec, _deg_spec, _b_spec],
    out_specs=_y_out_spec,
    out_shape=jax.ShapeDtypeStruct((NC, NP, H), jnp.float32),
)

_final = pl.pallas_call(
    _final_body,
    grid=(NP // BLK,),
    in_specs=[
        pl.BlockSpec((2, BLK, H), lambda i: (0, i, 0)),
        pl.BlockSpec((2, BLK, 16), lambda i: (0, i, 0)),
        pl.BlockSpec((1, D), lambda i: (0, 0)),
    ],
    out_specs=pl.BlockSpec((BLK, D), lambda i: (i, 0)),
    out_shape=jax.ShapeDtypeStruct((N, D), jnp.float32),
)


@jax.jit
def kernel(graph, nfeat, W1, b1, W2, b2, W3, b3):
    src = graph[0].astype(jnp.int32)
    dst = graph[1].astype(jnp.int32)
    dst_deg = dst.reshape(NC * NS, E // (NC * NS) // CH, CH)
    src_agg = src.reshape(NS, E // NS // CH, CH)
    dst_agg = dst.reshape(NS, E // NS // CH, CH)
    ones_c = jnp.ones((CH, 16), jnp.float32)
    zeros_c = jnp.zeros((RPT, 16), jnp.float32)
    x0 = jnp.pad(nfeat, ((0, NP - N), (0, 0)))

    deg = _deg_kernel(dst_deg, ones_c, zeros_c)

    xw = _mm1(x0, W1)
    y = _scale(xw, deg)
    z = _agg_kernel(y, src_agg, dst_agg)
    y = _mm_fused(z, W2, deg, b1.reshape(1, D))
    z = _agg_kernel(y, src_agg, dst_agg)
    y = _mm_fused(z, W3, deg, b2.reshape(1, D))
    z = _agg_kernel(y, src_agg, dst_agg)
    return _final(z, deg, b3.reshape(1, D))


# speedup vs baseline: 2.4146x; 1.0353x over previous
import functools
import jax
import jax.numpy as jnp
from jax import lax
from jax.experimental import pallas as pl
from jax.experimental.pallas import tpu as pltpu
from jax.experimental.pallas import tpu_sc as plsc

N = 10000
NP = 10240
E = 160000
D = 256
H = 128
NC, NS = 2, 16
CH = 125
RPT = NP // NS

_MESH = plsc.VectorSubcoreMesh(
    core_axis_name="c", subcore_axis_name="s", num_cores=NC, num_subcores=NS)


def _deg_body(dst_hbm, ones_hbm, zeros_hbm, out_hbm, idx_v, ones_v, acc, sem):
    c = lax.axis_index("c")
    s = lax.axis_index("s")
    w = c * NS + s
    r0 = s * RPT
    pltpu.sync_copy(zeros_hbm, acc.at[pl.ds(r0, RPT)])
    pltpu.sync_copy(dst_hbm.at[w], idx_v)
    pltpu.sync_copy(ones_hbm, ones_v)
    plsc.subcore_barrier()

    def body(j, carry):
        pltpu.sync_copy(ones_v, acc.at[idx_v.at[j]], add=True)
        return carry

    lax.fori_loop(0, dst_hbm.shape[1], body, 0)
    plsc.subcore_barrier()
    pltpu.sync_copy(acc.at[pl.ds(r0, RPT)], out_hbm.at[c].at[pl.ds(r0, RPT)])


_deg_kernel = functools.partial(
    pl.kernel,
    out_type=jax.ShapeDtypeStruct((NC, NP, 16), jnp.float32),
    mesh=_MESH,
    scratch_types=[
        pltpu.VMEM((E // (NC * NS) // CH, CH), jnp.int32),
        pltpu.VMEM((CH, 16), jnp.float32),
        pltpu.VMEM_SHARED((NP, 16), jnp.float32),
        pltpu.SemaphoreType.DMA,
    ],
)(_deg_body)


UNROLL = 16
NB = (E // NS // CH) // UNROLL


def _agg_body(y_hbm, src_hbm, dst_hbm, out_hbm, src_v, dst_v,
              rows0, rows1, acc, sg0, sg1, sd):
    c = lax.axis_index("c")
    s = lax.axis_index("s")
    r0 = s * RPT
    pltpu.sync_copy(y_hbm.at[c].at[pl.ds(r0, RPT)], acc.at[pl.ds(r0, RPT)])
    pltpu.sync_copy(src_hbm.at[s], src_v)
    plsc.subcore_barrier()

    rbufs = (rows0, rows1)
    gsems = (sg0, sg1)

    def gather(j, k):
        pltpu.async_copy(y_hbm.at[c].at[src_v.at[j]], rbufs[k % 2],
                         gsems[k % 2])

    def wait_gather(j, k):
        pltpu.make_async_copy(y_hbm.at[c].at[src_v.at[j]], rbufs[k % 2],
                              gsems[k % 2]).wait()

    def dst_half(g):
        return dst_v.at[pl.ds(lax.rem(g, 2) * UNROLL, UNROLL)]

    def dst_src(g):
        return dst_hbm.at[s].at[pl.ds(g * UNROLL, UNROLL)]

    pltpu.async_copy(dst_src(0), dst_half(0), sd)
    gather(0, 0)

    def body(g, carry):
        base = g * UNROLL
        pltpu.make_async_copy(dst_src(g), dst_half(g), sd).wait()

        @pl.when(g + 1 < NB)
        def _():
            pltpu.async_copy(dst_src(g + 1), dst_half(g + 1), sd)

        for k in range(UNROLL):
            j = base + k
            if k + 1 < UNROLL:
                gather(j + 1, k + 1)
            else:
                @pl.when(g + 1 < NB)
                def _():
                    gather(j + 1, k + 1)
            wait_gather(j, k)
            pltpu.sync_copy(rbufs[k % 2],
                            acc.at[dst_v.at[lax.rem(g, 2) * UNROLL + k]],
                            add=True)
        return carry

    lax.fori_loop(0, NB, body, 0)
    plsc.subcore_barrier()
    pltpu.sync_copy(acc.at[pl.ds(r0, RPT)], out_hbm.at[c].at[pl.ds(r0, RPT)])


_agg_kernel = functools.partial(
    pl.kernel,
    out_type=jax.ShapeDtypeStruct((NC, NP, H), jnp.float32),
    mesh=_MESH,
    scratch_types=[
        pltpu.VMEM((E // NS // CH, CH), jnp.int32),
        pltpu.VMEM((2 * UNROLL, CH), jnp.int32),
        pltpu.VMEM((CH, H), jnp.float32),
        pltpu.VMEM((CH, H), jnp.float32),
        pltpu.VMEM_SHARED((NP, H), jnp.float32),
        pltpu.SemaphoreType.DMA,
        pltpu.SemaphoreType.DMA,
        pltpu.SemaphoreType.DMA,
    ],
)(_agg_body)


BLK = 1024


def _dinv_of(deg_ref):
    return lax.rsqrt(deg_ref[0, :, :1] + deg_ref[1, :, :1])


def _mm1_body(x_ref, w_ref, y_ref):
    y_ref[0] = jnp.dot(x_ref[...], w_ref[...],
                       preferred_element_type=jnp.float32)


def _scale_body(z_ref, deg_ref, y_ref):
    y_ref[...] = _dinv_of(deg_ref) * z_ref[...]


def _mm_fused_body(z_ref, w_ref, deg_ref, b_ref, y_ref):
    dinv = _dinv_of(deg_ref)
    z = jnp.concatenate([z_ref[0], z_ref[1]], axis=-1)
    x = jnp.maximum(dinv * z + b_ref[...], 0.0)
    y_ref[0] = dinv * jnp.dot(x, w_ref[...],
                              preferred_element_type=jnp.float32)


def _final_body(z_ref, deg_ref, b_ref, out_ref):
    dinv = _dinv_of(deg_ref)
    z = jnp.concatenate([z_ref[0], z_ref[1]], axis=-1)
    h = dinv * z + b_ref[...]
    m = jnp.max(h, axis=-1, keepdims=True)
    e = jnp.exp(h - m)
    lse = m + jnp.log(jnp.sum(e, axis=-1, keepdims=True))
    out_ref[...] = h - lse


_deg_spec = pl.BlockSpec((2, BLK, 16), lambda i, h: (0, i, 0))
_w_spec = pl.BlockSpec((D, H), lambda i, h: (0, h))
_y_out_spec = pl.BlockSpec((1, BLK, H), lambda i, h: (h, i, 0))
_z_spec = pl.BlockSpec((2, BLK, H), lambda i, h: (0, i, 0))
_b_spec = pl.BlockSpec((1, D), lambda i, h: (0, 0))

_mm1 = pl.pallas_call(
    _mm1_body,
    grid=(NP // BLK, 2),
    in_specs=[pl.BlockSpec((BLK, D), lambda i, h: (i, 0)), _w_spec],
    out_specs=_y_out_spec,
    out_shape=jax.ShapeDtypeStruct((NC, NP, H), jnp.float32),
)

_scale = pl.pallas_call(
    _scale_body,
    grid=(NP // BLK, 2),
    in_specs=[pl.BlockSpec((1, BLK, H), lambda i, h: (h, i, 0)), _deg_spec],
    out_specs=_y_out_spec,
    out_shape=jax.ShapeDtypeStruct((NC, NP, H), jnp.float32),
)

_mm_fused = pl.pallas_call(
    _mm_fused_body,
    grid=(NP // BLK, 2),
    in_specs=[_z_spec, _w_spec, _deg_spec, _b_spec],
    out_specs=_y_out_spec,
    out_shape=jax.ShapeDtypeStruct((NC, NP, H), jnp.float32),
)

_final = pl.pallas_call(
    _final_body,
    grid=(NP // BLK,),
    in_specs=[
        pl.BlockSpec((2, BLK, H), lambda i: (0, i, 0)),
        pl.BlockSpec((2, BLK, 16), lambda i: (0, i, 0)),
        pl.BlockSpec((1, D), lambda i: (0, 0)),
    ],
    out_specs=pl.BlockSpec((BLK, D), lambda i: (i, 0)),
    out_shape=jax.ShapeDtypeStruct((N, D), jnp.float32),
)


@jax.jit
def kernel(graph, nfeat, W1, b1, W2, b2, W3, b3):
    src = graph[0].astype(jnp.int32)
    dst = graph[1].astype(jnp.int32)
    dst_deg = dst.reshape(NC * NS, E // (NC * NS) // CH, CH)
    src_agg = src.reshape(NS, E // NS // CH, CH)
    dst_agg = dst.reshape(NS, E // NS // CH, CH)
    ones_c = jnp.ones((CH, 16), jnp.float32)
    zeros_c = jnp.zeros((RPT, 16), jnp.float32)
    x0 = jnp.pad(nfeat, ((0, NP - N), (0, 0)))

    deg = _deg_kernel(dst_deg, ones_c, zeros_c)

    xw = _mm1(x0, W1)
    y = _scale(xw, deg)
    z = _agg_kernel(y, src_agg, dst_agg)
    y = _mm_fused(z, W2, deg, b1.reshape(1, D))
    z = _agg_kernel(y, src_agg, dst_agg)
    y = _mm_fused(z, W3, deg, b2.reshape(1, D))
    z = _agg_kernel(y, src_agg, dst_agg)
    return _final(z, deg, b3.reshape(1, D))
